# baseline (speedup 1.0000x reference)
import functools

import jax
import jax.numpy as jnp
from jax import lax
from jax.experimental import pallas as pl
from jax.experimental.pallas import tpu as pltpu
from jax.experimental.pallas import tpu_sc as plsc

N = 50000
D = 128
E = 600000
NC = 2
NS = 16
CHUNKS = 4
CW = 32
GROUP = 128
RPT = 296
EP = NS * RPT * GROUP
ACC_N = 50176
ROWS_T = ACC_N // NS
IDXB = 8
NB = RPT // IDXB
BN = 2000


def _mm_body(fu_ref, fi_ref, w_ref, b_ref, wh_ref):
    for e in range(2):
        f = fu_ref[...] if e == 0 else fi_ref[...]
        wh = jnp.dot(f, w_ref[e], preferred_element_type=jnp.float32)
        wh_ref[e] = (wh + b_ref[e][None, :]).astype(jnp.bfloat16)


def _make_wh(feat_user, feat_item, ws, bs):
    return pl.pallas_call(
        _mm_body,
        grid=(N // BN,),
        in_specs=[
            pl.BlockSpec((BN, D), lambda i: (i, 0)),
            pl.BlockSpec((BN, D), lambda i: (i, 0)),
            pl.BlockSpec((2, D, D), lambda i: (0, 0, 0)),
            pl.BlockSpec((2, D), lambda i: (0, 0)),
        ],
        out_specs=pl.BlockSpec((2, BN, D), lambda i: (0, i, 0)),
        out_shape=jax.ShapeDtypeStruct((2, N, D), jnp.bfloat16),
    )(feat_user, feat_item, ws, bs)


_MESH = plsc.VectorSubcoreMesh(core_axis_name="c", subcore_axis_name="s")


@functools.partial(
    pl.kernel,
    out_type=(
        jax.ShapeDtypeStruct((2, ACC_N, D), jnp.bfloat16),
        jax.ShapeDtypeStruct((2, ACC_N, D), jnp.bfloat16),
    ),
    mesh=_MESH,
    compiler_params=pltpu.CompilerParams(use_tc_tiling_on_sc=False),
    scratch_types=[
        pltpu.VMEM((2, IDXB, GROUP), jnp.int32),
        pltpu.VMEM((2, IDXB, GROUP), jnp.int32),
        pltpu.VMEM((IDXB, GROUP, CW), jnp.bfloat16),
        pltpu.VMEM((IDXB, GROUP, CW), jnp.bfloat16),
        pltpu.VMEM((GROUP, CW), jnp.bfloat16),
        pltpu.VMEM_SHARED((ACC_N, CW), jnp.bfloat16),
        pltpu.SemaphoreType.DMA,
        pltpu.SemaphoreType.DMA,
        pltpu.SemaphoreType.DMA,
    ],
)
def _sc_segment_sums(tabs, srcs4, dsts, zslab, ones_in,
                     sums_out, cnt_out,
                     src_idx, dst_idx, rows_a, rows_b, ones_v,
                     acc, gsem, ssem, isem):
    cid = lax.axis_index("c")
    sid = lax.axis_index("s")
    r0 = sid * RPT
    a0 = sid * ROWS_T
    acc_slab = acc.at[pl.ds(a0, ROWS_T)]
    rows = (rows_a, rows_b)

    pltpu.sync_copy(ones_in, ones_v)

    def idx_copies(b, par):
        base = cid * (EP // GROUP) + r0 + b * IDXB
        return (pltpu.make_async_copy(srcs4.at[pl.ds(base, IDXB)],
                                      src_idx.at[par], isem),
                pltpu.make_async_copy(dsts.at[pl.ds(base, IDXB)],
                                      dst_idx.at[par], isem))

    def idx_fire(b, par):
        for c in idx_copies(b, par):
            c.start()

    def idx_wait_transform(b, par, p):
        for c in idx_copies(b, par):
            c.wait()
        for j in range(IDXB):
            for k in range(GROUP // 16):
                sl = src_idx[par, j, pl.ds(k * 16, 16)]
                src_idx[par, j, pl.ds(k * 16, 16)] = sl * CHUNKS + p

    def dst_copy(b, par):
        base = cid * (EP // GROUP) + r0 + b * IDXB
        return pltpu.make_async_copy(dsts.at[pl.ds(base, IDXB)],
                                     dst_idx.at[par], isem)

    pltpu.sync_copy(zslab, acc_slab)
    plsc.subcore_barrier()
    dst_copy(0, 0).start()
    dst_copy(0, 0).wait()

    def cnt_blk(b, carry):
        par = lax.rem(b, 2)
        opar = lax.rem(b + 1, 2)
        for j in range(IDXB):
            pltpu.async_copy(ones_v, acc.at[dst_idx.at[par, j]], ssem,
                             add=True)

        @pl.when(b > 0)
        def _():
            for j in range(IDXB):
                pltpu.make_async_copy(ones_v, acc.at[dst_idx.at[opar, j]],
                                      ssem).wait()

        @pl.when(b < NB - 1)
        def _():
            dst_copy(b + 1, opar).start()
            dst_copy(b + 1, opar).wait()
        return carry

    lax.fori_loop(0, NB, cnt_blk, 0)
    parl = lax.rem(NB - 1, 2)
    for j in range(IDXB):
        pltpu.make_async_copy(ones_v, acc.at[dst_idx.at[parl, j]],
                              ssem).wait()
    plsc.subcore_barrier()
    for p8 in range(CHUNKS):
        pltpu.sync_copy(acc_slab,
                        cnt_out.at[cid, pl.ds(a0, ROWS_T), pl.ds(p8 * CW, CW)])

    def feat_pass(p, carry):
        pltpu.sync_copy(zslab, acc_slab)
        plsc.subcore_barrier()
        tab = tabs.at[cid]

        def g_copy(par, j, rbuf):
            return pltpu.make_async_copy(tab.at[src_idx.at[par, j]],
                                         rbuf.at[j], gsem)

        def s_copy(par, j, rbuf):
            return pltpu.make_async_copy(rbuf.at[j],
                                         acc.at[dst_idx.at[par, j]], ssem)

        idx_fire(0, 0)
        idx_wait_transform(0, 0, p)
        for j in range(IDXB):
            g_copy(0, j, rows_a).start()

        def blk(b, carry2):
            par = lax.rem(b, 2)
            for ri in range(2):
                rbuf = rows[ri]
                obuf = rows[1 - ri]

                @pl.when(par == ri)
                def _(b=b, ri=ri, rbuf=rbuf, obuf=obuf):
                    for j in range(IDXB):
                        g_copy(ri, j, rbuf).wait()

                    @pl.when(b > 0)
                    def _():
                        for j in range(IDXB):
                            s_copy(1 - ri, j, obuf).wait()

                    @pl.when(b < NB - 1)
                    def _():
                        idx_fire(b + 1, 1 - ri)

                    for j in range(IDXB):
                        s_copy(ri, j, rbuf).start(add=True)

                    @pl.when(b < NB - 1)
                    def _():
                        idx_wait_transform(b + 1, 1 - ri, p)
                        for j in range(IDXB):
                            g_copy(1 - ri, j, obuf).start()
            return carry2

        lax.fori_loop(0, NB, blk, 0)
        parl2 = lax.rem(NB - 1, 2)
        for ri in range(2):
            @pl.when(parl2 == ri)
            def _(ri=ri):
                for j in range(IDXB):
                    s_copy(ri, j, rows[ri]).wait()
        plsc.subcore_barrier()
        pltpu.sync_copy(
            acc_slab,
            sums_out.at[cid, pl.ds(a0, ROWS_T),
                        pl.ds(pl.multiple_of(p * CW, CW), CW)])
        return carry

    lax.fori_loop(0, CHUNKS, feat_pass, 0)


def _div_body(sums_ref, cnt_ref, hu_ref, hi_ref):
    s0 = sums_ref[0].astype(jnp.float32)
    s1 = sums_ref[1].astype(jnp.float32)
    c0 = jnp.maximum(cnt_ref[0].astype(jnp.float32), 1.0)
    c1 = jnp.maximum(cnt_ref[1].astype(jnp.float32), 1.0)
    hi_ref[...] = s0 / c0
    hu_ref[...] = s1 / c1


def _divide(sums, cnt):
    return pl.pallas_call(
        _div_body,
        grid=(N // BN,),
        in_specs=[
            pl.BlockSpec((2, BN, D), lambda i: (0, i, 0)),
            pl.BlockSpec((2, BN, D), lambda i: (0, i, 0)),
        ],
        out_specs=[
            pl.BlockSpec((BN, D), lambda i: (i, 0)),
            pl.BlockSpec((BN, D), lambda i: (i, 0)),
        ],
        out_shape=[
            jax.ShapeDtypeStruct((N, D), jnp.float32),
            jax.ShapeDtypeStruct((N, D), jnp.float32),
        ],
    )(sums, cnt)


def kernel(feat_user, feat_item, W_rates, b_rates, W_rated_by, b_rated_by,
           edge_index_rates, edge_index_rated_by):
    ws = jnp.stack([W_rates, W_rated_by])
    bs = jnp.stack([b_rates, b_rated_by])
    pad = jnp.zeros((EP - E,), jnp.int32)
    pad_dst = jnp.full((EP - E,), N, jnp.int32)
    srcs4 = jnp.concatenate([
        edge_index_rates[0], pad,
        edge_index_rated_by[0], pad,
    ]).reshape(2 * EP // GROUP, GROUP)
    dsts = jnp.concatenate([
        edge_index_rates[1], pad_dst,
        edge_index_rated_by[1], pad_dst,
    ]).reshape(2 * EP // GROUP, GROUP)
    zslab = jnp.zeros((ROWS_T, CW), jnp.bfloat16)
    ones_in = jnp.ones((GROUP, CW), jnp.bfloat16)

    wh = _make_wh(feat_user, feat_item, ws, bs)
    tabs = wh.reshape(2, N * CHUNKS, CW)
    sums, cnt = _sc_segment_sums(tabs, srcs4, dsts, zslab, ones_in)
    h_user, h_item = _divide(sums, cnt)
    return (h_user, h_item)


# speedup vs baseline: 1.0611x; 1.0611x over previous
import functools

import jax
import jax.numpy as jnp
from jax import lax
from jax.experimental import pallas as pl
from jax.experimental.pallas import tpu as pltpu
from jax.experimental.pallas import tpu_sc as plsc

N = 50000
D = 128
E = 600000
NC = 2
NS = 16
CHUNKS = 4
CW = 32
GROUP = 128
RPT = 296
EP = NS * RPT * GROUP
ACC_N = 50176
ROWS_T = ACC_N // NS
IDXB = 8
NB = RPT // IDXB
BN = 2000


PBN = 592


def _pack_body(er_ref, eb_ref, src_ref, dst_ref):
    e = pl.program_id(0)
    i = pl.program_id(1)
    pos = (i * (PBN * GROUP)
           + lax.broadcasted_iota(jnp.int32, (PBN, GROUP), 0) * GROUP
           + lax.broadcasted_iota(jnp.int32, (PBN, GROUP), 1))
    mask = pos < E
    sel = jnp.equal(e, 0)
    s = jnp.where(sel, er_ref[0], eb_ref[0]).reshape(PBN, GROUP)
    d = jnp.where(sel, er_ref[1], eb_ref[1]).reshape(PBN, GROUP)
    src_ref[...] = jnp.where(mask, s, 0)
    dst_ref[...] = jnp.where(mask, d, N)


def _pack_idx(edge_index_rates, edge_index_rated_by):
    nblk = (EP // GROUP) // PBN
    return pl.pallas_call(
        _pack_body,
        grid=(2, nblk),
        in_specs=[
            pl.BlockSpec((2, PBN * GROUP), lambda e, i: (0, i)),
            pl.BlockSpec((2, PBN * GROUP), lambda e, i: (0, i)),
        ],
        out_specs=[
            pl.BlockSpec((PBN, GROUP), lambda e, i: (e * ((EP // GROUP) // PBN) + i, 0)),
            pl.BlockSpec((PBN, GROUP), lambda e, i: (e * ((EP // GROUP) // PBN) + i, 0)),
        ],
        out_shape=[
            jax.ShapeDtypeStruct((2 * EP // GROUP, GROUP), jnp.int32),
            jax.ShapeDtypeStruct((2 * EP // GROUP, GROUP), jnp.int32),
        ],
    )(edge_index_rates, edge_index_rated_by)


def _mm_body(fu_ref, fi_ref, w_ref, b_ref, wh_ref):
    for e in range(2):
        f = fu_ref[...] if e == 0 else fi_ref[...]
        wh = jnp.dot(f, w_ref[e], preferred_element_type=jnp.float32)
        wh_ref[e] = (wh + b_ref[e][None, :]).astype(jnp.bfloat16)


def _make_wh(feat_user, feat_item, ws, bs):
    return pl.pallas_call(
        _mm_body,
        grid=(N // BN,),
        in_specs=[
            pl.BlockSpec((BN, D), lambda i: (i, 0)),
            pl.BlockSpec((BN, D), lambda i: (i, 0)),
            pl.BlockSpec((2, D, D), lambda i: (0, 0, 0)),
            pl.BlockSpec((2, D), lambda i: (0, 0)),
        ],
        out_specs=pl.BlockSpec((2, BN, D), lambda i: (0, i, 0)),
        out_shape=jax.ShapeDtypeStruct((2, N, D), jnp.bfloat16),
    )(feat_user, feat_item, ws, bs)


_MESH = plsc.VectorSubcoreMesh(core_axis_name="c", subcore_axis_name="s")


@functools.partial(
    pl.kernel,
    out_type=(
        jax.ShapeDtypeStruct((2, ACC_N, D), jnp.bfloat16),
        jax.ShapeDtypeStruct((2, ACC_N, D), jnp.bfloat16),
    ),
    mesh=_MESH,
    compiler_params=pltpu.CompilerParams(use_tc_tiling_on_sc=False),
    scratch_types=[
        pltpu.VMEM((2, IDXB, GROUP), jnp.int32),
        pltpu.VMEM((2, IDXB, GROUP), jnp.int32),
        pltpu.VMEM((IDXB, GROUP, CW), jnp.bfloat16),
        pltpu.VMEM((IDXB, GROUP, CW), jnp.bfloat16),
        pltpu.VMEM((GROUP, CW), jnp.bfloat16),
        pltpu.VMEM_SHARED((ACC_N, CW), jnp.bfloat16),
        pltpu.SemaphoreType.DMA,
        pltpu.SemaphoreType.DMA,
        pltpu.SemaphoreType.DMA,
    ],
)
def _sc_segment_sums(tabs, srcs4, dsts, zslab, ones_in,
                     sums_out, cnt_out,
                     src_idx, dst_idx, rows_a, rows_b, ones_v,
                     acc, gsem, ssem, isem):
    cid = lax.axis_index("c")
    sid = lax.axis_index("s")
    r0 = sid * RPT
    a0 = sid * ROWS_T
    acc_slab = acc.at[pl.ds(a0, ROWS_T)]
    rows = (rows_a, rows_b)

    pltpu.sync_copy(ones_in, ones_v)

    def idx_copies(b, par):
        base = cid * (EP // GROUP) + r0 + b * IDXB
        return (pltpu.make_async_copy(srcs4.at[pl.ds(base, IDXB)],
                                      src_idx.at[par], isem),
                pltpu.make_async_copy(dsts.at[pl.ds(base, IDXB)],
                                      dst_idx.at[par], isem))

    def idx_fire(b, par):
        for c in idx_copies(b, par):
            c.start()

    def idx_wait_transform(b, par, p):
        for c in idx_copies(b, par):
            c.wait()
        for j in range(IDXB):
            for k in range(GROUP // 16):
                sl = src_idx[par, j, pl.ds(k * 16, 16)]
                src_idx[par, j, pl.ds(k * 16, 16)] = sl * CHUNKS + p

    def dst_copy(b, par):
        base = cid * (EP // GROUP) + r0 + b * IDXB
        return pltpu.make_async_copy(dsts.at[pl.ds(base, IDXB)],
                                     dst_idx.at[par], isem)

    pltpu.sync_copy(zslab, acc_slab)
    plsc.subcore_barrier()
    dst_copy(0, 0).start()
    dst_copy(0, 0).wait()

    def cnt_blk(b, carry):
        par = lax.rem(b, 2)
        opar = lax.rem(b + 1, 2)
        for j in range(IDXB):
            pltpu.async_copy(ones_v, acc.at[dst_idx.at[par, j]], ssem,
                             add=True)

        @pl.when(b > 0)
        def _():
            for j in range(IDXB):
                pltpu.make_async_copy(ones_v, acc.at[dst_idx.at[opar, j]],
                                      ssem).wait()

        @pl.when(b < NB - 1)
        def _():
            dst_copy(b + 1, opar).start()
            dst_copy(b + 1, opar).wait()
        return carry

    lax.fori_loop(0, NB, cnt_blk, 0)
    parl = lax.rem(NB - 1, 2)
    for j in range(IDXB):
        pltpu.make_async_copy(ones_v, acc.at[dst_idx.at[parl, j]],
                              ssem).wait()
    plsc.subcore_barrier()
    for p8 in range(CHUNKS):
        pltpu.sync_copy(acc_slab,
                        cnt_out.at[cid, pl.ds(a0, ROWS_T), pl.ds(p8 * CW, CW)])

    def feat_pass(p, carry):
        pltpu.sync_copy(zslab, acc_slab)
        plsc.subcore_barrier()
        tab = tabs.at[cid]

        def g_copy(par, j, rbuf):
            return pltpu.make_async_copy(tab.at[src_idx.at[par, j]],
                                         rbuf.at[j], gsem)

        def s_copy(par, j, rbuf):
            return pltpu.make_async_copy(rbuf.at[j],
                                         acc.at[dst_idx.at[par, j]], ssem)

        idx_fire(0, 0)
        idx_wait_transform(0, 0, p)
        for j in range(IDXB):
            g_copy(0, j, rows_a).start()

        def blk(b, carry2):
            par = lax.rem(b, 2)
            for ri in range(2):
                rbuf = rows[ri]
                obuf = rows[1 - ri]

                @pl.when(par == ri)
                def _(b=b, ri=ri, rbuf=rbuf, obuf=obuf):
                    for j in range(IDXB):
                        g_copy(ri, j, rbuf).wait()

                    @pl.when(b > 0)
                    def _():
                        for j in range(IDXB):
                            s_copy(1 - ri, j, obuf).wait()

                    @pl.when(b < NB - 1)
                    def _():
                        idx_fire(b + 1, 1 - ri)

                    for j in range(IDXB):
                        s_copy(ri, j, rbuf).start(add=True)

                    @pl.when(b < NB - 1)
                    def _():
                        idx_wait_transform(b + 1, 1 - ri, p)
                        for j in range(IDXB):
                            g_copy(1 - ri, j, obuf).start()
            return carry2

        lax.fori_loop(0, NB, blk, 0)
        parl2 = lax.rem(NB - 1, 2)
        for ri in range(2):
            @pl.when(parl2 == ri)
            def _(ri=ri):
                for j in range(IDXB):
                    s_copy(ri, j, rows[ri]).wait()
        plsc.subcore_barrier()
        pltpu.sync_copy(
            acc_slab,
            sums_out.at[cid, pl.ds(a0, ROWS_T),
                        pl.ds(pl.multiple_of(p * CW, CW), CW)])
        return carry

    lax.fori_loop(0, CHUNKS, feat_pass, 0)


def _div_body(sums_ref, cnt_ref, hu_ref, hi_ref):
    s0 = sums_ref[0].astype(jnp.float32)
    s1 = sums_ref[1].astype(jnp.float32)
    c0 = jnp.maximum(cnt_ref[0].astype(jnp.float32), 1.0)
    c1 = jnp.maximum(cnt_ref[1].astype(jnp.float32), 1.0)
    hi_ref[...] = s0 / c0
    hu_ref[...] = s1 / c1


def _divide(sums, cnt):
    return pl.pallas_call(
        _div_body,
        grid=(N // BN,),
        in_specs=[
            pl.BlockSpec((2, BN, D), lambda i: (0, i, 0)),
            pl.BlockSpec((2, BN, D), lambda i: (0, i, 0)),
        ],
        out_specs=[
            pl.BlockSpec((BN, D), lambda i: (i, 0)),
            pl.BlockSpec((BN, D), lambda i: (i, 0)),
        ],
        out_shape=[
            jax.ShapeDtypeStruct((N, D), jnp.float32),
            jax.ShapeDtypeStruct((N, D), jnp.float32),
        ],
    )(sums, cnt)


def kernel(feat_user, feat_item, W_rates, b_rates, W_rated_by, b_rated_by,
           edge_index_rates, edge_index_rated_by):
    ws = jnp.stack([W_rates, W_rated_by])
    bs = jnp.stack([b_rates, b_rated_by])
    srcs4, dsts = _pack_idx(edge_index_rates, edge_index_rated_by)
    zslab = jnp.zeros((ROWS_T, CW), jnp.bfloat16)
    ones_in = jnp.ones((GROUP, CW), jnp.bfloat16)

    wh = _make_wh(feat_user, feat_item, ws, bs)
    tabs = wh.reshape(2, N * CHUNKS, CW)
    sums, cnt = _sc_segment_sums(tabs, srcs4, dsts, zslab, ones_in)
    h_user, h_item = _divide(sums, cnt)
    return (h_user, h_item)
